# Initial kernel scaffold; baseline (speedup 1.0000x reference)
import jax
import jax.numpy as jnp
from jax.experimental import pallas as pl


def kernel(indices, table):
    raise NotImplementedError("write your pallas kernel here")


# speedup vs baseline: 1.7973x; 1.7973x over previous
import functools

import jax
import jax.numpy as jnp
from jax import lax
from jax.experimental import pallas as pl
from jax.experimental.pallas import tpu as pltpu
from jax.experimental.pallas import tpu_sc as plsc

VOCAB = 1000000
EMB_DIM = 64

NC = 2
NS = 16
NW = NC * NS

SUB = 128
K = 4
CHUNK = SUB * K


def _emb_body(n_chunks, idx_hbm, table_hbm, out_hbm, idx_v, rows_v, gsem):
  wid = lax.axis_index("s") * NC + lax.axis_index("c")
  base = wid * (n_chunks * K)

  def chunk(i, _):
    row = base + i * K
    pltpu.sync_copy(idx_hbm.at[pl.ds(row, K)], idx_v)
    copies = []
    for j in range(K):
      copies.append(
          pltpu.async_copy(
              table_hbm.at[idx_v.at[j]],
              rows_v.at[pl.ds(j * SUB, SUB)],
              gsem,
          )
      )
    for c in copies:
      c.wait()
    pltpu.sync_copy(rows_v, out_hbm.at[pl.ds(row * SUB, CHUNK)])
    return 0

  lax.fori_loop(0, n_chunks, chunk, 0)


def kernel(indices, table):
  B = indices.size
  assert B % (NW * CHUNK) == 0
  n_chunks = B // (NW * CHUNK)
  idx_flat = indices.reshape(B // SUB, SUB).astype(jnp.int32)

  mesh = plsc.VectorSubcoreMesh(core_axis_name="c", subcore_axis_name="s")
  grab = pl.kernel(
      functools.partial(_emb_body, n_chunks),
      out_type=jax.ShapeDtypeStruct((B, EMB_DIM), jnp.float32),
      mesh=mesh,
      scratch_types=[
          pltpu.VMEM((K, SUB), jnp.int32),
          pltpu.VMEM((CHUNK, EMB_DIM), jnp.float32),
          pltpu.SemaphoreType.DMA,
      ],
      compiler_params=pltpu.CompilerParams(use_tc_tiling_on_sc=False),
  )
  out = grab(idx_flat, table)
  return out.reshape(indices.shape + (EMB_DIM,))


# speedup vs baseline: 1.8739x; 1.0426x over previous
import functools

import jax
import jax.numpy as jnp
from jax import lax
from jax.experimental import pallas as pl
from jax.experimental.pallas import tpu as pltpu
from jax.experimental.pallas import tpu_sc as plsc

VOCAB = 1000000
EMB_DIM = 64

NC = 2
NS = 16
NW = NC * NS

SUB = 128
K = 4
CHUNK = SUB * K
NBUF = 2


def _emb_body(n_chunks, idx_hbm, table_hbm, out_hbm, idx_v, rows_v,
              g0, g1, w0, w1):
  gsem = (g0, g1)
  wsem = (w0, w1)
  wid = lax.axis_index("s") * NC + lax.axis_index("c")
  idx_rows = n_chunks * K
  base = wid * idx_rows
  out_base = base * SUB

  pltpu.sync_copy(idx_hbm.at[pl.ds(base, idx_rows)], idx_v)

  def fire_gather(c, b):
    for j in range(K):
      pltpu.async_copy(
          table_hbm.at[idx_v.at[c * K + j]],
          rows_v.at[b, pl.ds(j * SUB, SUB)],
          gsem[b],
      )

  def wait_gather(b):
    pltpu.make_async_copy(
        table_hbm.at[pl.ds(0, CHUNK)], rows_v.at[b], gsem[b]).wait()

  def fire_write(c, b):
    pltpu.async_copy(
        rows_v.at[b], out_hbm.at[pl.ds(out_base + c * CHUNK, CHUNK)], wsem[b])

  def wait_write(b):
    pltpu.make_async_copy(
        rows_v.at[b], out_hbm.at[pl.ds(0, CHUNK)], wsem[b]).wait()

  for b in range(NBUF):
    fire_gather(b, b)

  n_outer = n_chunks // NBUF

  def outer(g, _):
    for b in range(NBUF):
      c = g * NBUF + b
      wait_gather(b)
      fire_write(c, b)

      @pl.when(g < n_outer - 1)
      def _():
        wait_write(b)
        fire_gather(c + NBUF, b)

    return 0

  lax.fori_loop(0, n_outer, outer, 0)
  for b in range(NBUF):
    wait_write(b)


def kernel(indices, table):
  B = indices.size
  assert B % (NW * CHUNK * NBUF) == 0
  n_chunks = B // (NW * CHUNK)
  idx_flat = indices.reshape(B // SUB, SUB).astype(jnp.int32)

  mesh = plsc.VectorSubcoreMesh(core_axis_name="c", subcore_axis_name="s")
  grab = pl.kernel(
      functools.partial(_emb_body, n_chunks),
      out_type=jax.ShapeDtypeStruct((B, EMB_DIM), jnp.float32),
      mesh=mesh,
      scratch_types=[
          pltpu.VMEM((n_chunks * K, SUB), jnp.int32),
          pltpu.VMEM((NBUF, CHUNK, EMB_DIM), jnp.float32),
          pltpu.SemaphoreType.DMA,
          pltpu.SemaphoreType.DMA,
          pltpu.SemaphoreType.DMA,
          pltpu.SemaphoreType.DMA,
      ],
      compiler_params=pltpu.CompilerParams(use_tc_tiling_on_sc=False),
  )
  out = grab(idx_flat, table)
  return out.reshape(indices.shape + (EMB_DIM,))
